# baseline (speedup 1.0000x reference)
import functools

import jax
import jax.numpy as jnp
from jax import lax
from jax.experimental import pallas as pl
from jax.experimental.pallas import tpu as pltpu
from jax.experimental.pallas import tpu_sc as plsc

N = 4096
K = 16
EPS = 0.1
NEG_INF = float("-inf")


def _topk_body(s_ref, vals_ref, idx_ref):
    v = s_ref[...]
    c = v.shape[1]
    rows = jax.lax.broadcasted_iota(jnp.int32, (N, c), 0)
    for k in range(K):
        m = jnp.max(v, axis=0, keepdims=True)
        cand = jnp.where(v == m, rows, N)
        am = jnp.min(cand, axis=0, keepdims=True)
        vals_ref[k : k + 1, :] = m
        idx_ref[k : k + 1, :] = am
        v = jnp.where(rows == am, NEG_INF, v)


def _topk(S):
    C = 256
    grid = (N // C,)
    return pl.pallas_call(
        _topk_body,
        grid=grid,
        in_specs=[pl.BlockSpec((N, C), lambda j: (0, j))],
        out_specs=[
            pl.BlockSpec((K, C), lambda j: (0, j)),
            pl.BlockSpec((K, C), lambda j: (0, j)),
        ],
        out_shape=[
            jax.ShapeDtypeStruct((K, N), jnp.float32),
            jax.ShapeDtypeStruct((K, N), jnp.int32),
        ],
        compiler_params=pltpu.CompilerParams(
            dimension_semantics=("arbitrary",)
        ),
    )(S)


def _build_body(vals_ref, idx_ref, h_ref, degd_ref, binv_ref):
    rb = pl.program_id(0)
    r = h_ref.shape[0]
    mv = (vals_ref[...] > EPS).astype(jnp.float32)
    iv = idx_ref[...]
    rows = jax.lax.broadcasted_iota(jnp.int32, (r, 1), 0) + rb * r
    acc = jnp.zeros((r, N), jnp.float32)
    for k in range(K):
        acc = acc + jnp.where(iv[k : k + 1, :] == rows, mv[k : k + 1, :], 0.0)
    h_ref[...] = acc
    degd_ref[...] = jnp.sum(acc, axis=1, keepdims=True)

    @pl.when(rb == 0)
    def _():
        degb = jnp.sum(mv, axis=0, keepdims=True)
        binv_ref[...] = jnp.where(degb > 0, 1.0 / jnp.maximum(degb, 1e-9), 0.0)


def _build(vals, idx):
    R = 512
    grid = (N // R,)
    return pl.pallas_call(
        _build_body,
        grid=grid,
        in_specs=[
            pl.BlockSpec((K, N), lambda i: (0, 0)),
            pl.BlockSpec((K, N), lambda i: (0, 0)),
        ],
        out_specs=[
            pl.BlockSpec((R, N), lambda i: (i, 0)),
            pl.BlockSpec((R, 1), lambda i: (i, 0)),
            pl.BlockSpec((1, N), lambda i: (0, 0)),
        ],
        out_shape=[
            jax.ShapeDtypeStruct((N, N), jnp.float32),
            jax.ShapeDtypeStruct((N, 1), jnp.float32),
            jax.ShapeDtypeStruct((1, N), jnp.float32),
        ],
        compiler_params=pltpu.CompilerParams(
            dimension_semantics=("arbitrary",)
        ),
    )(vals, idx)


def _agg_t_body(h_ref, x_ref, out_ref):
    kb = pl.program_id(1)
    prod = jax.lax.dot_general(
        h_ref[...], x_ref[...], (((0,), (0,)), ((), ())),
        preferred_element_type=jnp.float32,
    )

    @pl.when(kb == 0)
    def _():
        out_ref[...] = prod

    @pl.when(kb != 0)
    def _():
        out_ref[...] += prod


def _agg_t(H, x):
    F = x.shape[1]
    J = 1024
    R = 1024
    grid = (N // J, N // R)
    return pl.pallas_call(
        _agg_t_body,
        grid=grid,
        in_specs=[
            pl.BlockSpec((R, J), lambda j, k: (k, j)),
            pl.BlockSpec((R, F), lambda j, k: (k, 0)),
        ],
        out_specs=pl.BlockSpec((J, F), lambda j, k: (j, 0)),
        out_shape=jax.ShapeDtypeStruct((N, F), jnp.float32),
        compiler_params=pltpu.CompilerParams(
            dimension_semantics=("parallel", "arbitrary")
        ),
    )(H, x)


def _scatter_body(h_ref, he_ref, binv_ref, degd_ref, bias_ref, out_ref, *,
                  nk, final_tanh):
    kb = pl.program_id(1)
    hb = h_ref[...] * binv_ref[...]
    prod = jnp.dot(hb, he_ref[...], preferred_element_type=jnp.float32)

    @pl.when(kb == 0)
    def _():
        out_ref[...] = prod

    @pl.when(kb != 0)
    def _():
        out_ref[...] += prod

    @pl.when(kb == nk - 1)
    def _():
        dv = degd_ref[...]
        dinv = jnp.where(dv > 0, 1.0 / jnp.maximum(dv, 1e-9), 0.0)
        r = out_ref[...] * dinv + bias_ref[...]
        out_ref[...] = jnp.tanh(r) if final_tanh else r


def _scatter(H, he, binv, degd, bias, final_tanh):
    F = he.shape[1]
    R = 1024
    J = 1024
    nk = N // J
    grid = (N // R, nk)
    return pl.pallas_call(
        functools.partial(_scatter_body, nk=nk, final_tanh=final_tanh),
        grid=grid,
        in_specs=[
            pl.BlockSpec((R, J), lambda i, k: (i, k)),
            pl.BlockSpec((J, F), lambda i, k: (k, 0)),
            pl.BlockSpec((1, J), lambda i, k: (0, k)),
            pl.BlockSpec((R, 1), lambda i, k: (i, 0)),
            pl.BlockSpec((1, F), lambda i, k: (0, 0)),
        ],
        out_specs=pl.BlockSpec((R, F), lambda i, k: (i, 0)),
        out_shape=jax.ShapeDtypeStruct((N, F), jnp.float32),
        compiler_params=pltpu.CompilerParams(
            dimension_semantics=("parallel", "arbitrary")
        ),
    )(H, he, binv, degd, bias)


def _mlp_body(z_ref, w1_ref, b1_ref, w2_ref, out_ref):
    mid = jnp.dot(z_ref[...], w1_ref[...], preferred_element_type=jnp.float32)
    mid = jnp.maximum(mid + b1_ref[...], 0.0)
    out_ref[...] = jnp.dot(mid, w2_ref[...], preferred_element_type=jnp.float32)


def _mlp(z, W1, b1, W2):
    IN_F, HID = W1.shape
    CODE = W2.shape[1]
    R = 512
    grid = (N // R,)
    return pl.pallas_call(
        _mlp_body,
        grid=grid,
        in_specs=[
            pl.BlockSpec((R, IN_F), lambda i: (i, 0)),
            pl.BlockSpec((IN_F, HID), lambda i: (0, 0)),
            pl.BlockSpec((1, HID), lambda i: (0, 0)),
            pl.BlockSpec((HID, CODE), lambda i: (0, 0)),
        ],
        out_specs=pl.BlockSpec((R, CODE), lambda i: (i, 0)),
        out_shape=jax.ShapeDtypeStruct((N, CODE), jnp.float32),
        compiler_params=pltpu.CompilerParams(
            dimension_semantics=("arbitrary",)
        ),
    )(z, W1, b1, W2)


def _prep_body(vals_ref, idx_ref, safe_ref, binv_ref):
    mv = vals_ref[...] > EPS
    safe = jnp.where(mv, idx_ref[...], N)
    safe_ref[...] = jnp.transpose(safe, (1, 0))
    degb = jnp.sum(mv.astype(jnp.float32), axis=0, keepdims=True)
    binv_ref[...] = jnp.where(degb > 0, 1.0 / jnp.maximum(degb, 1e-9), 0.0)


def _prep(vals, idx):
    C = 512
    grid = (N // C,)
    return pl.pallas_call(
        _prep_body,
        grid=grid,
        in_specs=[
            pl.BlockSpec((K, C), lambda j: (0, j)),
            pl.BlockSpec((K, C), lambda j: (0, j)),
        ],
        out_specs=[
            pl.BlockSpec((C, K), lambda j: (j, 0)),
            pl.BlockSpec((1, C), lambda j: (0, j)),
        ],
        out_shape=[
            jax.ShapeDtypeStruct((N, K), jnp.int32),
            jax.ShapeDtypeStruct((1, N), jnp.float32),
        ],
        compiler_params=pltpu.CompilerParams(
            dimension_semantics=("arbitrary",)
        ),
    )(vals, idx)


_NW = 32
_JPW = N // _NW


def _sc_conv(t_pad, safeT, binv, zeros, F):
    mesh = plsc.VectorSubcoreMesh(core_axis_name="c", subcore_axis_name="s")

    @functools.partial(
        pl.kernel,
        out_type=[
            jax.ShapeDtypeStruct((N, F), jnp.float32),
            jax.ShapeDtypeStruct((N, F), jnp.float32),
        ],
        mesh=mesh,
        scratch_types=[
            pltpu.VMEM((_JPW, K), jnp.int32),
            pltpu.VMEM((1, _JPW + 16), jnp.float32),
            pltpu.VMEM((K, F), jnp.float32),
            pltpu.VMEM((K, F), jnp.float32),
            pltpu.VMEM((N // 16, F), jnp.float32),
            pltpu.VMEM_SHARED((N + 1, F), jnp.float32),
        ],
    )
    def body(t_hbm, safe_hbm, binv_hbm, zeros_hbm, out_a, out_b,
             safe_v, binv_v, rows_v, sbuf_v, stage_v, acc_sh):
        cid = lax.axis_index("c")
        sid = lax.axis_index("s")
        j0 = (cid * 16 + sid) * _JPW

        @pl.when(sid == 0)
        def _():
            pltpu.sync_copy(zeros_hbm, acc_sh)

        pltpu.sync_copy(safe_hbm.at[pl.ds(j0, _JPW)], safe_v)
        pltpu.sync_copy(binv_hbm.at[:, pl.ds(j0, _JPW)],
                        binv_v.at[:, pl.ds(0, _JPW)])
        plsc.subcore_barrier()

        def step(jj, carry):
            pltpu.sync_copy(t_hbm.at[safe_v.at[jj]], rows_v)
            bv = binv_v[0, pl.ds(jj, 16)][0]
            for c in range(F // 16):
                sl = pl.ds(c * 16, 16)
                acc = rows_v[0, sl]
                for kk in range(1, K):
                    acc = acc + rows_v[kk, sl]
                acc = acc * bv
                for kk in range(K):
                    sbuf_v[kk, sl] = acc
            pltpu.sync_copy(sbuf_v, acc_sh.at[safe_v.at[jj]], add=True)
            return carry

        lax.fori_loop(0, _JPW, step, 0)
        plsc.subcore_barrier()
        r0 = sid * (N // 16)
        pltpu.sync_copy(acc_sh.at[pl.ds(r0, N // 16)], stage_v)

        @pl.when(cid == 0)
        def _():
            pltpu.sync_copy(stage_v, out_a.at[pl.ds(r0, N // 16)])

        @pl.when(cid == 1)
        def _():
            pltpu.sync_copy(stage_v, out_b.at[pl.ds(r0, N // 16)])

    return body(t_pad, safeT, binv, zeros)


def _finish_body(a_ref, b_ref, degd_ref, bias_ref, out_ref):
    F = out_ref.shape[1]
    dv = degd_ref[...]
    dinv = jnp.where(dv > 0, 1.0 / jnp.maximum(dv, 1e-9), 0.0)
    s = a_ref[:, :F] + b_ref[:, :F]
    out_ref[...] = jnp.tanh(s * dinv + bias_ref[...])


def _finish(za, zb, degd, bias):
    Fp = za.shape[1]
    F = bias.shape[1]
    R = 1024
    grid = (N // R,)
    return pl.pallas_call(
        _finish_body,
        grid=grid,
        in_specs=[
            pl.BlockSpec((R, Fp), lambda i: (i, 0)),
            pl.BlockSpec((R, Fp), lambda i: (i, 0)),
            pl.BlockSpec((R, 1), lambda i: (i, 0)),
            pl.BlockSpec((1, F), lambda i: (0, 0)),
        ],
        out_specs=pl.BlockSpec((R, F), lambda i: (i, 0)),
        out_shape=jax.ShapeDtypeStruct((N, F), jnp.float32),
        compiler_params=pltpu.CompilerParams(
            dimension_semantics=("arbitrary",)
        ),
    )(za, zb, degd, bias)


def kernel(x, S, W1, b1, W2, b2):
    vals, idx = _topk(S)
    H, degd, binv = _build(vals, idx)
    zero_b = jnp.zeros((1, x.shape[1]), jnp.float32)
    he = _agg_t(H, x)
    z = _scatter(H, he, binv, degd, zero_b, final_tanh=False)
    t = _mlp(z, W1, b1.reshape(1, -1), W2)
    t_pad = jnp.pad(t, ((0, 1), (0, 128 - t.shape[1])))
    zeros = jnp.zeros((N + 1, 128), jnp.float32)
    safeT, binv_p = _prep(vals, idx)
    z2a, z2b = _sc_conv(t_pad, safeT, binv_p, zeros, 128)
    code = _finish(z2a, z2b, degd, b2.reshape(1, -1))
    return code


# speedup vs baseline: 1.1002x; 1.1002x over previous
import functools

import jax
import jax.numpy as jnp
from jax import lax
from jax.experimental import pallas as pl
from jax.experimental.pallas import tpu as pltpu
from jax.experimental.pallas import tpu_sc as plsc

N = 4096
K = 16
EPS = 0.1
NEG_INF = float("-inf")


def _topk_body(s_ref, vals_ref, idx_ref):
    v = s_ref[...]
    c = v.shape[1]
    rows = jax.lax.broadcasted_iota(jnp.int32, (N, c), 0)
    for k in range(K):
        m = jnp.max(v, axis=0, keepdims=True)
        cand = jnp.where(v == m, rows, N)
        am = jnp.min(cand, axis=0, keepdims=True)
        vals_ref[k : k + 1, :] = m
        idx_ref[k : k + 1, :] = am
        v = jnp.where(rows == am, NEG_INF, v)


def _topk(S):
    C = 256
    grid = (N // C,)
    return pl.pallas_call(
        _topk_body,
        grid=grid,
        in_specs=[pl.BlockSpec((N, C), lambda j: (0, j))],
        out_specs=[
            pl.BlockSpec((K, C), lambda j: (0, j)),
            pl.BlockSpec((K, C), lambda j: (0, j)),
        ],
        out_shape=[
            jax.ShapeDtypeStruct((K, N), jnp.float32),
            jax.ShapeDtypeStruct((K, N), jnp.int32),
        ],
        compiler_params=pltpu.CompilerParams(
            dimension_semantics=("arbitrary",)
        ),
    )(S)


def _build_body(vals_ref, idx_ref, h_ref, degd_ref, binv_ref):
    rb = pl.program_id(0)
    r = h_ref.shape[0]
    mv = (vals_ref[...] > EPS).astype(jnp.float32)
    iv = idx_ref[...]
    rows = jax.lax.broadcasted_iota(jnp.int32, (r, 1), 0) + rb * r
    acc = jnp.zeros((r, N), jnp.float32)
    for k in range(K):
        acc = acc + jnp.where(iv[k : k + 1, :] == rows, mv[k : k + 1, :], 0.0)
    h_ref[...] = acc
    degd_ref[...] = jnp.sum(acc, axis=1, keepdims=True)

    @pl.when(rb == 0)
    def _():
        degb = jnp.sum(mv, axis=0, keepdims=True)
        binv_ref[...] = jnp.where(degb > 0, 1.0 / jnp.maximum(degb, 1e-9), 0.0)


def _build(vals, idx):
    R = 512
    grid = (N // R,)
    return pl.pallas_call(
        _build_body,
        grid=grid,
        in_specs=[
            pl.BlockSpec((K, N), lambda i: (0, 0)),
            pl.BlockSpec((K, N), lambda i: (0, 0)),
        ],
        out_specs=[
            pl.BlockSpec((R, N), lambda i: (i, 0)),
            pl.BlockSpec((R, 1), lambda i: (i, 0)),
            pl.BlockSpec((1, N), lambda i: (0, 0)),
        ],
        out_shape=[
            jax.ShapeDtypeStruct((N, N), jnp.float32),
            jax.ShapeDtypeStruct((N, 1), jnp.float32),
            jax.ShapeDtypeStruct((1, N), jnp.float32),
        ],
        compiler_params=pltpu.CompilerParams(
            dimension_semantics=("arbitrary",)
        ),
    )(vals, idx)


def _agg_t_body(h_ref, x_ref, out_ref):
    kb = pl.program_id(1)
    prod = jax.lax.dot_general(
        h_ref[...], x_ref[...], (((0,), (0,)), ((), ())),
        preferred_element_type=jnp.float32,
    )

    @pl.when(kb == 0)
    def _():
        out_ref[...] = prod

    @pl.when(kb != 0)
    def _():
        out_ref[...] += prod


def _agg_t(H, x):
    F = x.shape[1]
    J = 1024
    R = 1024
    grid = (N // J, N // R)
    return pl.pallas_call(
        _agg_t_body,
        grid=grid,
        in_specs=[
            pl.BlockSpec((R, J), lambda j, k: (k, j)),
            pl.BlockSpec((R, F), lambda j, k: (k, 0)),
        ],
        out_specs=pl.BlockSpec((J, F), lambda j, k: (j, 0)),
        out_shape=jax.ShapeDtypeStruct((N, F), jnp.float32),
        compiler_params=pltpu.CompilerParams(
            dimension_semantics=("parallel", "arbitrary")
        ),
    )(H, x)


def _scatter_body(h_ref, he_ref, binv_ref, degd_ref, bias_ref, out_ref, *,
                  nk, final_tanh):
    kb = pl.program_id(1)
    hb = h_ref[...] * binv_ref[...]
    prod = jnp.dot(hb, he_ref[...], preferred_element_type=jnp.float32)

    @pl.when(kb == 0)
    def _():
        out_ref[...] = prod

    @pl.when(kb != 0)
    def _():
        out_ref[...] += prod

    @pl.when(kb == nk - 1)
    def _():
        dv = degd_ref[...]
        dinv = jnp.where(dv > 0, 1.0 / jnp.maximum(dv, 1e-9), 0.0)
        r = out_ref[...] * dinv + bias_ref[...]
        out_ref[...] = jnp.tanh(r) if final_tanh else r


def _scatter(H, he, binv, degd, bias, final_tanh):
    F = he.shape[1]
    R = 1024
    J = 1024
    nk = N // J
    grid = (N // R, nk)
    return pl.pallas_call(
        functools.partial(_scatter_body, nk=nk, final_tanh=final_tanh),
        grid=grid,
        in_specs=[
            pl.BlockSpec((R, J), lambda i, k: (i, k)),
            pl.BlockSpec((J, F), lambda i, k: (k, 0)),
            pl.BlockSpec((1, J), lambda i, k: (0, k)),
            pl.BlockSpec((R, 1), lambda i, k: (i, 0)),
            pl.BlockSpec((1, F), lambda i, k: (0, 0)),
        ],
        out_specs=pl.BlockSpec((R, F), lambda i, k: (i, 0)),
        out_shape=jax.ShapeDtypeStruct((N, F), jnp.float32),
        compiler_params=pltpu.CompilerParams(
            dimension_semantics=("parallel", "arbitrary")
        ),
    )(H, he, binv, degd, bias)


def _mlp_body(z_ref, w1_ref, b1_ref, w2_ref, out_ref):
    mid = jnp.dot(z_ref[...], w1_ref[...], preferred_element_type=jnp.float32)
    mid = jnp.maximum(mid + b1_ref[...], 0.0)
    out_ref[...] = jnp.dot(mid, w2_ref[...], preferred_element_type=jnp.float32)


def _mlp(z, W1, b1, W2):
    IN_F, HID = W1.shape
    CODE = W2.shape[1]
    R = 512
    grid = (N // R,)
    return pl.pallas_call(
        _mlp_body,
        grid=grid,
        in_specs=[
            pl.BlockSpec((R, IN_F), lambda i: (i, 0)),
            pl.BlockSpec((IN_F, HID), lambda i: (0, 0)),
            pl.BlockSpec((1, HID), lambda i: (0, 0)),
            pl.BlockSpec((HID, CODE), lambda i: (0, 0)),
        ],
        out_specs=pl.BlockSpec((R, CODE), lambda i: (i, 0)),
        out_shape=jax.ShapeDtypeStruct((N, CODE), jnp.float32),
        compiler_params=pltpu.CompilerParams(
            dimension_semantics=("arbitrary",)
        ),
    )(z, W1, b1, W2)


def _prep_body(vals_ref, idx_ref, safe_ref, binv_ref):
    mv = vals_ref[...] > EPS
    safe = jnp.where(mv, idx_ref[...], N)
    safe_ref[...] = jnp.transpose(safe, (1, 0))
    degb = jnp.sum(mv.astype(jnp.float32), axis=0, keepdims=True)
    binv_ref[...] = jnp.where(degb > 0, 1.0 / jnp.maximum(degb, 1e-9), 0.0)


def _prep(vals, idx):
    C = 512
    grid = (N // C,)
    return pl.pallas_call(
        _prep_body,
        grid=grid,
        in_specs=[
            pl.BlockSpec((K, C), lambda j: (0, j)),
            pl.BlockSpec((K, C), lambda j: (0, j)),
        ],
        out_specs=[
            pl.BlockSpec((C, K), lambda j: (j, 0)),
            pl.BlockSpec((1, C), lambda j: (0, j)),
        ],
        out_shape=[
            jax.ShapeDtypeStruct((N, K), jnp.int32),
            jax.ShapeDtypeStruct((1, N), jnp.float32),
        ],
        compiler_params=pltpu.CompilerParams(
            dimension_semantics=("arbitrary",)
        ),
    )(vals, idx)


_NW = 32
_JPW = N // _NW


_JPG = 8
_NG = _JPW // _JPG


def _sc_conv(t_pad, safe_flat, safe_grp, binv, zeros, F):
    mesh = plsc.VectorSubcoreMesh(core_axis_name="c", subcore_axis_name="s")

    @functools.partial(
        pl.kernel,
        out_type=[
            jax.ShapeDtypeStruct((N, F), jnp.float32),
            jax.ShapeDtypeStruct((N, F), jnp.float32),
        ],
        mesh=mesh,
        scratch_types=[
            pltpu.VMEM((_JPW * K,), jnp.int32),
            pltpu.VMEM((_NG, _JPG * K), jnp.int32),
            pltpu.VMEM((1, _JPW + 16), jnp.float32),
            pltpu.VMEM((_JPG * K, F), jnp.float32),
            pltpu.VMEM((_JPG * K, F), jnp.float32),
            pltpu.VMEM((N // 16, F), jnp.float32),
            pltpu.VMEM_SHARED((N + 1, F), jnp.float32),
        ],
    )
    def body(t_hbm, safe1_hbm, safe2_hbm, binv_hbm, zeros_hbm, out_a, out_b,
             safe1_v, safe2_v, binv_v, rows_v, sbuf_v, stage_v, acc_sh):
        cid = lax.axis_index("c")
        sid = lax.axis_index("s")
        wid = cid * 16 + sid
        j0 = wid * _JPW

        @pl.when(sid == 0)
        def _():
            pltpu.sync_copy(zeros_hbm, acc_sh)

        pltpu.sync_copy(safe1_hbm.at[pl.ds(j0 * K, _JPW * K)], safe1_v)
        pltpu.sync_copy(safe2_hbm.at[pl.ds(wid * _NG, _NG)], safe2_v)
        pltpu.sync_copy(binv_hbm.at[:, pl.ds(j0, _JPW)],
                        binv_v.at[:, pl.ds(0, _JPW)])
        plsc.subcore_barrier()

        def step(g, carry):
            pltpu.sync_copy(t_hbm.at[safe1_v.at[pl.ds(g * (_JPG * K), _JPG * K)]],
                            rows_v)
            for q in range(_JPG):
                bv = binv_v[0, pl.ds(g * _JPG + q, 16)][0]
                for c in range(F // 16):
                    sl = pl.ds(c * 16, 16)
                    acc = rows_v[q * K, sl]
                    for kk in range(1, K):
                        acc = acc + rows_v[q * K + kk, sl]
                    acc = acc * bv
                    for kk in range(K):
                        sbuf_v[q * K + kk, sl] = acc
            pltpu.sync_copy(sbuf_v, acc_sh.at[safe2_v.at[g]], add=True)
            return carry

        lax.fori_loop(0, _NG, step, 0)
        plsc.subcore_barrier()
        r0 = sid * (N // 16)
        pltpu.sync_copy(acc_sh.at[pl.ds(r0, N // 16)], stage_v)

        @pl.when(cid == 0)
        def _():
            pltpu.sync_copy(stage_v, out_a.at[pl.ds(r0, N // 16)])

        @pl.when(cid == 1)
        def _():
            pltpu.sync_copy(stage_v, out_b.at[pl.ds(r0, N // 16)])

    return body(t_pad, safe_flat, safe_grp, binv, zeros)


def _finish_body(a_ref, b_ref, degd_ref, bias_ref, out_ref):
    F = out_ref.shape[1]
    dv = degd_ref[...]
    dinv = jnp.where(dv > 0, 1.0 / jnp.maximum(dv, 1e-9), 0.0)
    s = a_ref[:, :F] + b_ref[:, :F]
    out_ref[...] = jnp.tanh(s * dinv + bias_ref[...])


def _finish(za, zb, degd, bias):
    Fp = za.shape[1]
    F = bias.shape[1]
    R = 1024
    grid = (N // R,)
    return pl.pallas_call(
        _finish_body,
        grid=grid,
        in_specs=[
            pl.BlockSpec((R, Fp), lambda i: (i, 0)),
            pl.BlockSpec((R, Fp), lambda i: (i, 0)),
            pl.BlockSpec((R, 1), lambda i: (i, 0)),
            pl.BlockSpec((1, F), lambda i: (0, 0)),
        ],
        out_specs=pl.BlockSpec((R, F), lambda i: (i, 0)),
        out_shape=jax.ShapeDtypeStruct((N, F), jnp.float32),
        compiler_params=pltpu.CompilerParams(
            dimension_semantics=("arbitrary",)
        ),
    )(za, zb, degd, bias)


def kernel(x, S, W1, b1, W2, b2):
    vals, idx = _topk(S)
    H, degd, binv = _build(vals, idx)
    zero_b = jnp.zeros((1, x.shape[1]), jnp.float32)
    he = _agg_t(H, x)
    z = _scatter(H, he, binv, degd, zero_b, final_tanh=False)
    t = _mlp(z, W1, b1.reshape(1, -1), W2)
    t_pad = jnp.pad(t, ((0, 1), (0, 128 - t.shape[1])))
    zeros = jnp.zeros((N + 1, 128), jnp.float32)
    safeT, binv_p = _prep(vals, idx)
    safe_flat = safeT.reshape(-1)
    safe_grp = safeT.reshape(N * K // (_JPG * K), _JPG * K)
    z2a, z2b = _sc_conv(t_pad, safe_flat, safe_grp, binv_p, zeros, 128)
    code = _finish(z2a, z2b, degd, b2.reshape(1, -1))
    return code


# speedup vs baseline: 1.2731x; 1.1572x over previous
import functools

import jax
import jax.numpy as jnp
from jax import lax
from jax.experimental import pallas as pl
from jax.experimental.pallas import tpu as pltpu
from jax.experimental.pallas import tpu_sc as plsc

N = 4096
K = 16
EPS = 0.1
NEG_INF = float("-inf")


def _topk_body(s_ref, idx_ref, cnt_ref):
    v = s_ref[...]
    c = v.shape[1]
    rows = jax.lax.broadcasted_iota(jnp.int32, (N, c), 0)
    cnt_ref[...] = jnp.sum((v > EPS).astype(jnp.float32), axis=0,
                           keepdims=True)
    am = jnp.full((1, c), -1, jnp.int32)
    for k in range(K):
        v = jnp.where(rows == am, NEG_INF, v)
        am = jnp.argmax(v, axis=0).astype(jnp.int32).reshape(1, c)
        idx_ref[k : k + 1, :] = am


def _topk(S):
    C = 256
    grid = (N // C,)
    return pl.pallas_call(
        _topk_body,
        grid=grid,
        in_specs=[pl.BlockSpec((N, C), lambda j: (0, j))],
        out_specs=[
            pl.BlockSpec((K, C), lambda j: (0, j)),
            pl.BlockSpec((1, C), lambda j: (0, j)),
        ],
        out_shape=[
            jax.ShapeDtypeStruct((K, N), jnp.int32),
            jax.ShapeDtypeStruct((1, N), jnp.float32),
        ],
        compiler_params=pltpu.CompilerParams(
            dimension_semantics=("arbitrary",)
        ),
    )(S)


def _build_body(cnt_ref, idx_ref, h_ref, degd_ref, binv_ref):
    rb = pl.program_id(0)
    r = h_ref.shape[0]
    ks = jax.lax.broadcasted_iota(jnp.int32, (K, 1), 0).astype(jnp.float32)
    mv = (ks < cnt_ref[...]).astype(jnp.float32)
    iv = idx_ref[...]
    rows = jax.lax.broadcasted_iota(jnp.int32, (r, 1), 0) + rb * r
    acc = jnp.zeros((r, N), jnp.float32)
    for k in range(K):
        acc = acc + jnp.where(iv[k : k + 1, :] == rows, mv[k : k + 1, :], 0.0)
    h_ref[...] = acc
    degd_ref[...] = jnp.sum(acc, axis=1, keepdims=True)

    @pl.when(rb == 0)
    def _():
        degb = jnp.sum(mv, axis=0, keepdims=True)
        binv_ref[...] = jnp.where(degb > 0, 1.0 / jnp.maximum(degb, 1e-9), 0.0)


def _build(cnt, idx):
    R = 512
    grid = (N // R,)
    return pl.pallas_call(
        _build_body,
        grid=grid,
        in_specs=[
            pl.BlockSpec((1, N), lambda i: (0, 0)),
            pl.BlockSpec((K, N), lambda i: (0, 0)),
        ],
        out_specs=[
            pl.BlockSpec((R, N), lambda i: (i, 0)),
            pl.BlockSpec((R, 1), lambda i: (i, 0)),
            pl.BlockSpec((1, N), lambda i: (0, 0)),
        ],
        out_shape=[
            jax.ShapeDtypeStruct((N, N), jnp.float32),
            jax.ShapeDtypeStruct((N, 1), jnp.float32),
            jax.ShapeDtypeStruct((1, N), jnp.float32),
        ],
        compiler_params=pltpu.CompilerParams(
            dimension_semantics=("arbitrary",)
        ),
    )(cnt, idx)


def _agg_t_body(h_ref, x_ref, out_ref):
    kb = pl.program_id(1)
    prod = jax.lax.dot_general(
        h_ref[...], x_ref[...], (((0,), (0,)), ((), ())),
        preferred_element_type=jnp.float32,
    )

    @pl.when(kb == 0)
    def _():
        out_ref[...] = prod

    @pl.when(kb != 0)
    def _():
        out_ref[...] += prod


def _agg_t(H, x):
    F = x.shape[1]
    J = 1024
    R = 1024
    grid = (N // J, N // R)
    return pl.pallas_call(
        _agg_t_body,
        grid=grid,
        in_specs=[
            pl.BlockSpec((R, J), lambda j, k: (k, j)),
            pl.BlockSpec((R, F), lambda j, k: (k, 0)),
        ],
        out_specs=pl.BlockSpec((J, F), lambda j, k: (j, 0)),
        out_shape=jax.ShapeDtypeStruct((N, F), jnp.float32),
        compiler_params=pltpu.CompilerParams(
            dimension_semantics=("parallel", "arbitrary")
        ),
    )(H, x)


def _scatter_body(h_ref, he_ref, binv_ref, degd_ref, bias_ref, out_ref, *,
                  nk, final_tanh):
    kb = pl.program_id(1)
    hb = h_ref[...] * binv_ref[...]
    prod = jnp.dot(hb, he_ref[...], preferred_element_type=jnp.float32)

    @pl.when(kb == 0)
    def _():
        out_ref[...] = prod

    @pl.when(kb != 0)
    def _():
        out_ref[...] += prod

    @pl.when(kb == nk - 1)
    def _():
        dv = degd_ref[...]
        dinv = jnp.where(dv > 0, 1.0 / jnp.maximum(dv, 1e-9), 0.0)
        r = out_ref[...] * dinv + bias_ref[...]
        out_ref[...] = jnp.tanh(r) if final_tanh else r


def _scatter(H, he, binv, degd, bias, final_tanh):
    F = he.shape[1]
    R = 1024
    J = 1024
    nk = N // J
    grid = (N // R, nk)
    return pl.pallas_call(
        functools.partial(_scatter_body, nk=nk, final_tanh=final_tanh),
        grid=grid,
        in_specs=[
            pl.BlockSpec((R, J), lambda i, k: (i, k)),
            pl.BlockSpec((J, F), lambda i, k: (k, 0)),
            pl.BlockSpec((1, J), lambda i, k: (0, k)),
            pl.BlockSpec((R, 1), lambda i, k: (i, 0)),
            pl.BlockSpec((1, F), lambda i, k: (0, 0)),
        ],
        out_specs=pl.BlockSpec((R, F), lambda i, k: (i, 0)),
        out_shape=jax.ShapeDtypeStruct((N, F), jnp.float32),
        compiler_params=pltpu.CompilerParams(
            dimension_semantics=("parallel", "arbitrary")
        ),
    )(H, he, binv, degd, bias)


def _mlp_body(z_ref, w1_ref, b1_ref, w2_ref, out_ref):
    mid = jnp.dot(z_ref[...], w1_ref[...], preferred_element_type=jnp.float32)
    mid = jnp.maximum(mid + b1_ref[...], 0.0)
    out_ref[...] = jnp.dot(mid, w2_ref[...], preferred_element_type=jnp.float32)


def _mlp(z, W1, b1, W2):
    IN_F, HID = W1.shape
    CODE = W2.shape[1]
    R = 512
    grid = (N // R,)
    return pl.pallas_call(
        _mlp_body,
        grid=grid,
        in_specs=[
            pl.BlockSpec((R, IN_F), lambda i: (i, 0)),
            pl.BlockSpec((IN_F, HID), lambda i: (0, 0)),
            pl.BlockSpec((1, HID), lambda i: (0, 0)),
            pl.BlockSpec((HID, CODE), lambda i: (0, 0)),
        ],
        out_specs=pl.BlockSpec((R, CODE), lambda i: (i, 0)),
        out_shape=jax.ShapeDtypeStruct((N, CODE), jnp.float32),
        compiler_params=pltpu.CompilerParams(
            dimension_semantics=("arbitrary",)
        ),
    )(z, W1, b1, W2)


def _prep_body(cnt_ref, idx_ref, safe_ref, binv_ref):
    ks = jax.lax.broadcasted_iota(jnp.int32, (K, 1), 0).astype(jnp.float32)
    mv = ks < cnt_ref[...]
    safe = jnp.where(mv, idx_ref[...], N)
    safe_ref[...] = jnp.transpose(safe, (1, 0))
    degb = jnp.sum(mv.astype(jnp.float32), axis=0, keepdims=True)
    binv_ref[...] = jnp.where(degb > 0, 1.0 / jnp.maximum(degb, 1e-9), 0.0)


def _prep(cnt, idx):
    C = 512
    grid = (N // C,)
    return pl.pallas_call(
        _prep_body,
        grid=grid,
        in_specs=[
            pl.BlockSpec((1, C), lambda j: (0, j)),
            pl.BlockSpec((K, C), lambda j: (0, j)),
        ],
        out_specs=[
            pl.BlockSpec((C, K), lambda j: (j, 0)),
            pl.BlockSpec((1, C), lambda j: (0, j)),
        ],
        out_shape=[
            jax.ShapeDtypeStruct((N, K), jnp.int32),
            jax.ShapeDtypeStruct((1, N), jnp.float32),
        ],
        compiler_params=pltpu.CompilerParams(
            dimension_semantics=("arbitrary",)
        ),
    )(cnt, idx)


_NW = 32
_JPW = N // _NW


_JPG = 8
_NG = _JPW // _JPG


def _sc_conv(t_pad, safe_flat, safe_grp, binv, zeros, F):
    mesh = plsc.VectorSubcoreMesh(core_axis_name="c", subcore_axis_name="s")

    @functools.partial(
        pl.kernel,
        out_type=[
            jax.ShapeDtypeStruct((N, F), jnp.float32),
            jax.ShapeDtypeStruct((N, F), jnp.float32),
        ],
        mesh=mesh,
        scratch_types=[
            pltpu.VMEM((_JPW * K,), jnp.int32),
            pltpu.VMEM((_NG, _JPG * K), jnp.int32),
            pltpu.VMEM((1, _JPW + 16), jnp.float32),
            pltpu.VMEM((_JPG * K, F), jnp.float32),
            pltpu.VMEM((_JPG * K, F), jnp.float32),
            pltpu.VMEM((N // 16, F), jnp.float32),
            pltpu.VMEM_SHARED((N + 1, F), jnp.float32),
        ],
    )
    def body(t_hbm, safe1_hbm, safe2_hbm, binv_hbm, zeros_hbm, out_a, out_b,
             safe1_v, safe2_v, binv_v, rows_v, sbuf_v, stage_v, acc_sh):
        cid = lax.axis_index("c")
        sid = lax.axis_index("s")
        wid = cid * 16 + sid
        j0 = wid * _JPW

        @pl.when(sid == 0)
        def _():
            pltpu.sync_copy(zeros_hbm, acc_sh)

        pltpu.sync_copy(safe1_hbm.at[pl.ds(j0 * K, _JPW * K)], safe1_v)
        pltpu.sync_copy(safe2_hbm.at[pl.ds(wid * _NG, _NG)], safe2_v)
        pltpu.sync_copy(binv_hbm.at[:, pl.ds(j0, _JPW)],
                        binv_v.at[:, pl.ds(0, _JPW)])
        plsc.subcore_barrier()

        def step(g, carry):
            pltpu.sync_copy(t_hbm.at[safe1_v.at[pl.ds(g * (_JPG * K), _JPG * K)]],
                            rows_v)
            for q in range(_JPG):
                bv = binv_v[0, pl.ds(g * _JPG + q, 16)][0]
                for c in range(F // 16):
                    sl = pl.ds(c * 16, 16)
                    acc = rows_v[q * K, sl]
                    for kk in range(1, K):
                        acc = acc + rows_v[q * K + kk, sl]
                    acc = acc * bv
                    for kk in range(K):
                        sbuf_v[q * K + kk, sl] = acc
            pltpu.sync_copy(sbuf_v, acc_sh.at[safe2_v.at[g]], add=True)
            return carry

        lax.fori_loop(0, _NG, step, 0)
        plsc.subcore_barrier()
        r0 = sid * (N // 16)
        pltpu.sync_copy(acc_sh.at[pl.ds(r0, N // 16)], stage_v)

        @pl.when(cid == 0)
        def _():
            pltpu.sync_copy(stage_v, out_a.at[pl.ds(r0, N // 16)])

        @pl.when(cid == 1)
        def _():
            pltpu.sync_copy(stage_v, out_b.at[pl.ds(r0, N // 16)])

    return body(t_pad, safe_flat, safe_grp, binv, zeros)


def _finish_body(a_ref, b_ref, degd_ref, bias_ref, out_ref):
    F = out_ref.shape[1]
    dv = degd_ref[...]
    dinv = jnp.where(dv > 0, 1.0 / jnp.maximum(dv, 1e-9), 0.0)
    s = a_ref[:, :F] + b_ref[:, :F]
    out_ref[...] = jnp.tanh(s * dinv + bias_ref[...])


def _finish(za, zb, degd, bias):
    Fp = za.shape[1]
    F = bias.shape[1]
    R = 1024
    grid = (N // R,)
    return pl.pallas_call(
        _finish_body,
        grid=grid,
        in_specs=[
            pl.BlockSpec((R, Fp), lambda i: (i, 0)),
            pl.BlockSpec((R, Fp), lambda i: (i, 0)),
            pl.BlockSpec((R, 1), lambda i: (i, 0)),
            pl.BlockSpec((1, F), lambda i: (0, 0)),
        ],
        out_specs=pl.BlockSpec((R, F), lambda i: (i, 0)),
        out_shape=jax.ShapeDtypeStruct((N, F), jnp.float32),
        compiler_params=pltpu.CompilerParams(
            dimension_semantics=("arbitrary",)
        ),
    )(za, zb, degd, bias)


def kernel(x, S, W1, b1, W2, b2):
    idx, cnt = _topk(S)
    H, degd, binv = _build(cnt, idx)
    zero_b = jnp.zeros((1, x.shape[1]), jnp.float32)
    he = _agg_t(H, x)
    z = _scatter(H, he, binv, degd, zero_b, final_tanh=False)
    t = _mlp(z, W1, b1.reshape(1, -1), W2)
    t_pad = jnp.pad(t, ((0, 1), (0, 128 - t.shape[1])))
    zeros = jnp.zeros((N + 1, 128), jnp.float32)
    safeT, binv_p = _prep(cnt, idx)
    safe_flat = safeT.reshape(-1)
    safe_grp = safeT.reshape(N * K // (_JPG * K), _JPG * K)
    z2a, z2b = _sc_conv(t_pad, safe_flat, safe_grp, binv_p, zeros, 128)
    code = _finish(z2a, z2b, degd, b2.reshape(1, -1))
    return code


# speedup vs baseline: 1.4551x; 1.1429x over previous
import functools

import jax
import jax.numpy as jnp
from jax import lax
from jax.experimental import pallas as pl
from jax.experimental.pallas import tpu as pltpu
from jax.experimental.pallas import tpu_sc as plsc

N = 4096
K = 16
EPS = 0.1
NEG_INF = float("-inf")


def _topk_body(s_ref, idx_ref, cnt_ref):
    v = s_ref[...]
    c = v.shape[1]
    rows = jax.lax.broadcasted_iota(jnp.int32, (N, c), 0)
    cnt_ref[...] = jnp.sum((v > EPS).astype(jnp.float32), axis=0,
                           keepdims=True)
    am = jnp.full((1, c), -1, jnp.int32)
    for k in range(K):
        v = jnp.where(rows == am, NEG_INF, v)
        am = jnp.argmax(v, axis=0).astype(jnp.int32).reshape(1, c)
        idx_ref[k : k + 1, :] = am


def _topk(S):
    C = 256
    grid = (N // C,)
    return pl.pallas_call(
        _topk_body,
        grid=grid,
        in_specs=[pl.BlockSpec((N, C), lambda j: (0, j))],
        out_specs=[
            pl.BlockSpec((K, C), lambda j: (0, j)),
            pl.BlockSpec((1, C), lambda j: (0, j)),
        ],
        out_shape=[
            jax.ShapeDtypeStruct((K, N), jnp.int32),
            jax.ShapeDtypeStruct((1, N), jnp.float32),
        ],
        compiler_params=pltpu.CompilerParams(
            dimension_semantics=("arbitrary",)
        ),
    )(S)


def _build_body(cnt_ref, idx_ref, h_ref, degd_ref, binv_ref):
    rb = pl.program_id(0)
    r = h_ref.shape[0]
    ks = jax.lax.broadcasted_iota(jnp.int32, (K, 1), 0).astype(jnp.float32)
    mv = ks < cnt_ref[...]
    mvb = mv.astype(jnp.bfloat16)
    iv = idx_ref[...].astype(jnp.int16)
    rows = (jax.lax.broadcasted_iota(jnp.int32, (r, 1), 0) + rb * r
            ).astype(jnp.int16)
    zero = jnp.zeros((r, N), jnp.bfloat16)
    acc = zero
    for k in range(K):
        acc = acc + jnp.where(iv[k : k + 1, :] == rows, mvb[k : k + 1, :],
                              zero)
    h_ref[...] = acc
    degd_ref[...] = jnp.sum(acc.astype(jnp.float32), axis=1, keepdims=True)

    @pl.when(rb == 0)
    def _():
        degb = jnp.sum(mv.astype(jnp.float32), axis=0, keepdims=True)
        binv_ref[...] = jnp.where(degb > 0, 1.0 / jnp.maximum(degb, 1e-9), 0.0)


def _build(cnt, idx):
    R = 512
    grid = (N // R,)
    return pl.pallas_call(
        _build_body,
        grid=grid,
        in_specs=[
            pl.BlockSpec((1, N), lambda i: (0, 0)),
            pl.BlockSpec((K, N), lambda i: (0, 0)),
        ],
        out_specs=[
            pl.BlockSpec((R, N), lambda i: (i, 0)),
            pl.BlockSpec((R, 1), lambda i: (i, 0)),
            pl.BlockSpec((1, N), lambda i: (0, 0)),
        ],
        out_shape=[
            jax.ShapeDtypeStruct((N, N), jnp.bfloat16),
            jax.ShapeDtypeStruct((N, 1), jnp.float32),
            jax.ShapeDtypeStruct((1, N), jnp.float32),
        ],
        compiler_params=pltpu.CompilerParams(
            dimension_semantics=("arbitrary",)
        ),
    )(cnt, idx)


def _agg_t_body(h_ref, x_ref, out_ref):
    kb = pl.program_id(1)
    prod = jax.lax.dot_general(
        h_ref[...].astype(jnp.float32), x_ref[...], (((0,), (0,)), ((), ())),
        preferred_element_type=jnp.float32,
    )

    @pl.when(kb == 0)
    def _():
        out_ref[...] = prod

    @pl.when(kb != 0)
    def _():
        out_ref[...] += prod


def _agg_t(H, x):
    F = x.shape[1]
    J = 1024
    R = 1024
    grid = (N // J, N // R)
    return pl.pallas_call(
        _agg_t_body,
        grid=grid,
        in_specs=[
            pl.BlockSpec((R, J), lambda j, k: (k, j)),
            pl.BlockSpec((R, F), lambda j, k: (k, 0)),
        ],
        out_specs=pl.BlockSpec((J, F), lambda j, k: (j, 0)),
        out_shape=jax.ShapeDtypeStruct((N, F), jnp.float32),
        compiler_params=pltpu.CompilerParams(
            dimension_semantics=("parallel", "arbitrary")
        ),
    )(H, x)


def _scatter_body(h_ref, he_ref, binv_ref, degd_ref, bias_ref, out_ref, *,
                  nk, final_tanh):
    kb = pl.program_id(1)
    hb = h_ref[...].astype(jnp.float32) * binv_ref[...]
    prod = jnp.dot(hb, he_ref[...], preferred_element_type=jnp.float32)

    @pl.when(kb == 0)
    def _():
        out_ref[...] = prod

    @pl.when(kb != 0)
    def _():
        out_ref[...] += prod

    @pl.when(kb == nk - 1)
    def _():
        dv = degd_ref[...]
        dinv = jnp.where(dv > 0, 1.0 / jnp.maximum(dv, 1e-9), 0.0)
        r = out_ref[...] * dinv + bias_ref[...]
        out_ref[...] = jnp.tanh(r) if final_tanh else r


def _scatter(H, he, binv, degd, bias, final_tanh):
    F = he.shape[1]
    R = 1024
    J = 1024
    nk = N // J
    grid = (N // R, nk)
    return pl.pallas_call(
        functools.partial(_scatter_body, nk=nk, final_tanh=final_tanh),
        grid=grid,
        in_specs=[
            pl.BlockSpec((R, J), lambda i, k: (i, k)),
            pl.BlockSpec((J, F), lambda i, k: (k, 0)),
            pl.BlockSpec((1, J), lambda i, k: (0, k)),
            pl.BlockSpec((R, 1), lambda i, k: (i, 0)),
            pl.BlockSpec((1, F), lambda i, k: (0, 0)),
        ],
        out_specs=pl.BlockSpec((R, F), lambda i, k: (i, 0)),
        out_shape=jax.ShapeDtypeStruct((N, F), jnp.float32),
        compiler_params=pltpu.CompilerParams(
            dimension_semantics=("parallel", "arbitrary")
        ),
    )(H, he, binv, degd, bias)


def _mlp_body(z_ref, w1_ref, b1_ref, w2_ref, out_ref):
    mid = jnp.dot(z_ref[...], w1_ref[...], preferred_element_type=jnp.float32)
    mid = jnp.maximum(mid + b1_ref[...], 0.0)
    out_ref[...] = jnp.dot(mid, w2_ref[...], preferred_element_type=jnp.float32)


def _mlp(z, W1, b1, W2):
    IN_F, HID = W1.shape
    CODE = W2.shape[1]
    R = 512
    grid = (N // R,)
    return pl.pallas_call(
        _mlp_body,
        grid=grid,
        in_specs=[
            pl.BlockSpec((R, IN_F), lambda i: (i, 0)),
            pl.BlockSpec((IN_F, HID), lambda i: (0, 0)),
            pl.BlockSpec((1, HID), lambda i: (0, 0)),
            pl.BlockSpec((HID, CODE), lambda i: (0, 0)),
        ],
        out_specs=pl.BlockSpec((R, CODE), lambda i: (i, 0)),
        out_shape=jax.ShapeDtypeStruct((N, CODE), jnp.float32),
        compiler_params=pltpu.CompilerParams(
            dimension_semantics=("arbitrary",)
        ),
    )(z, W1, b1, W2)


def _prep_body(cnt_ref, idx_ref, safe_ref, binv_ref):
    ks = jax.lax.broadcasted_iota(jnp.int32, (K, 1), 0).astype(jnp.float32)
    mv = ks < cnt_ref[...]
    safe = jnp.where(mv, idx_ref[...], N)
    safe_ref[...] = jnp.transpose(safe, (1, 0))
    degb = jnp.sum(mv.astype(jnp.float32), axis=0, keepdims=True)
    binv_ref[...] = jnp.where(degb > 0, 1.0 / jnp.maximum(degb, 1e-9), 0.0)


def _prep(cnt, idx):
    C = 512
    grid = (N // C,)
    return pl.pallas_call(
        _prep_body,
        grid=grid,
        in_specs=[
            pl.BlockSpec((1, C), lambda j: (0, j)),
            pl.BlockSpec((K, C), lambda j: (0, j)),
        ],
        out_specs=[
            pl.BlockSpec((C, K), lambda j: (j, 0)),
            pl.BlockSpec((1, C), lambda j: (0, j)),
        ],
        out_shape=[
            jax.ShapeDtypeStruct((N, K), jnp.int32),
            jax.ShapeDtypeStruct((1, N), jnp.float32),
        ],
        compiler_params=pltpu.CompilerParams(
            dimension_semantics=("arbitrary",)
        ),
    )(cnt, idx)


_NW = 32
_JPW = N // _NW


_JPG = 8
_NG = _JPW // _JPG


def _sc_conv(t_pad, safe_flat, safe_grp, binv, zeros, F):
    mesh = plsc.VectorSubcoreMesh(core_axis_name="c", subcore_axis_name="s")

    @functools.partial(
        pl.kernel,
        out_type=[
            jax.ShapeDtypeStruct((N, F), jnp.float32),
            jax.ShapeDtypeStruct((N, F), jnp.float32),
        ],
        mesh=mesh,
        scratch_types=[
            pltpu.VMEM((_JPW * K,), jnp.int32),
            pltpu.VMEM((_NG, _JPG * K), jnp.int32),
            pltpu.VMEM((1, _JPW + 16), jnp.float32),
            pltpu.VMEM((_JPG * K, F), jnp.float32),
            pltpu.VMEM((_JPG * K, F), jnp.float32),
            pltpu.VMEM((N // 16, F), jnp.float32),
            pltpu.VMEM_SHARED((N + 1, F), jnp.float32),
        ],
    )
    def body(t_hbm, safe1_hbm, safe2_hbm, binv_hbm, zeros_hbm, out_a, out_b,
             safe1_v, safe2_v, binv_v, rows_v, sbuf_v, stage_v, acc_sh):
        cid = lax.axis_index("c")
        sid = lax.axis_index("s")
        wid = cid * 16 + sid
        j0 = wid * _JPW

        @pl.when(sid == 0)
        def _():
            pltpu.sync_copy(zeros_hbm, acc_sh)

        pltpu.sync_copy(safe1_hbm.at[pl.ds(j0 * K, _JPW * K)], safe1_v)
        pltpu.sync_copy(safe2_hbm.at[pl.ds(wid * _NG, _NG)], safe2_v)
        pltpu.sync_copy(binv_hbm.at[:, pl.ds(j0, _JPW)],
                        binv_v.at[:, pl.ds(0, _JPW)])
        plsc.subcore_barrier()

        def step(g, carry):
            pltpu.sync_copy(t_hbm.at[safe1_v.at[pl.ds(g * (_JPG * K), _JPG * K)]],
                            rows_v)
            for q in range(_JPG):
                bv = binv_v[0, pl.ds(g * _JPG + q, 16)][0]
                for c in range(F // 16):
                    sl = pl.ds(c * 16, 16)
                    acc = rows_v[q * K, sl]
                    for kk in range(1, K):
                        acc = acc + rows_v[q * K + kk, sl]
                    acc = acc * bv
                    for kk in range(K):
                        sbuf_v[q * K + kk, sl] = acc
            pltpu.sync_copy(sbuf_v, acc_sh.at[safe2_v.at[g]], add=True)
            return carry

        lax.fori_loop(0, _NG, step, 0)
        plsc.subcore_barrier()
        r0 = sid * (N // 16)
        pltpu.sync_copy(acc_sh.at[pl.ds(r0, N // 16)], stage_v)

        @pl.when(cid == 0)
        def _():
            pltpu.sync_copy(stage_v, out_a.at[pl.ds(r0, N // 16)])

        @pl.when(cid == 1)
        def _():
            pltpu.sync_copy(stage_v, out_b.at[pl.ds(r0, N // 16)])

    return body(t_pad, safe_flat, safe_grp, binv, zeros)


def _finish_body(a_ref, b_ref, degd_ref, bias_ref, out_ref):
    F = out_ref.shape[1]
    dv = degd_ref[...]
    dinv = jnp.where(dv > 0, 1.0 / jnp.maximum(dv, 1e-9), 0.0)
    s = a_ref[:, :F] + b_ref[:, :F]
    out_ref[...] = jnp.tanh(s * dinv + bias_ref[...])


def _finish(za, zb, degd, bias):
    Fp = za.shape[1]
    F = bias.shape[1]
    R = 1024
    grid = (N // R,)
    return pl.pallas_call(
        _finish_body,
        grid=grid,
        in_specs=[
            pl.BlockSpec((R, Fp), lambda i: (i, 0)),
            pl.BlockSpec((R, Fp), lambda i: (i, 0)),
            pl.BlockSpec((R, 1), lambda i: (i, 0)),
            pl.BlockSpec((1, F), lambda i: (0, 0)),
        ],
        out_specs=pl.BlockSpec((R, F), lambda i: (i, 0)),
        out_shape=jax.ShapeDtypeStruct((N, F), jnp.float32),
        compiler_params=pltpu.CompilerParams(
            dimension_semantics=("arbitrary",)
        ),
    )(za, zb, degd, bias)


def kernel(x, S, W1, b1, W2, b2):
    idx, cnt = _topk(S)
    H, degd, binv = _build(cnt, idx)
    zero_b = jnp.zeros((1, x.shape[1]), jnp.float32)
    he = _agg_t(H, x)
    z = _scatter(H, he, binv, degd, zero_b, final_tanh=False)
    t = _mlp(z, W1, b1.reshape(1, -1), W2)
    t_pad = jnp.pad(t, ((0, 1), (0, 128 - t.shape[1])))
    zeros = jnp.zeros((N + 1, 128), jnp.float32)
    safeT, binv_p = _prep(cnt, idx)
    safe_flat = safeT.reshape(-1)
    safe_grp = safeT.reshape(N * K // (_JPG * K), _JPG * K)
    z2a, z2b = _sc_conv(t_pad, safe_flat, safe_grp, binv_p, zeros, 128)
    code = _finish(z2a, z2b, degd, b2.reshape(1, -1))
    return code


# speedup vs baseline: 1.4566x; 1.0011x over previous
import functools

import jax
import jax.numpy as jnp
from jax import lax
from jax.experimental import pallas as pl
from jax.experimental.pallas import tpu as pltpu
from jax.experimental.pallas import tpu_sc as plsc

N = 4096
K = 16
EPS = 0.1
NEG_INF = float("-inf")


def _topk_body(s_ref, idx_ref, cnt_ref):
    v = s_ref[...]
    c = v.shape[1]
    rows = jax.lax.broadcasted_iota(jnp.int32, (N, c), 0)
    cnt_ref[...] = jnp.sum((v > EPS).astype(jnp.float32), axis=0,
                           keepdims=True)
    am = jnp.full((1, c), -1, jnp.int32)
    for k in range(K):
        v = jnp.where(rows == am, NEG_INF, v)
        am = jnp.argmax(v, axis=0).astype(jnp.int32).reshape(1, c)
        idx_ref[k : k + 1, :] = am


def _topk(S):
    C = 256
    grid = (N // C,)
    return pl.pallas_call(
        _topk_body,
        grid=grid,
        in_specs=[pl.BlockSpec((N, C), lambda j: (0, j))],
        out_specs=[
            pl.BlockSpec((K, C), lambda j: (0, j)),
            pl.BlockSpec((1, C), lambda j: (0, j)),
        ],
        out_shape=[
            jax.ShapeDtypeStruct((K, N), jnp.int32),
            jax.ShapeDtypeStruct((1, N), jnp.float32),
        ],
        compiler_params=pltpu.CompilerParams(
            dimension_semantics=("arbitrary",)
        ),
    )(S)


def _build_body(cnt_ref, idx_ref, h_ref, degd_ref, binv_ref):
    rb = pl.program_id(0)
    r = h_ref.shape[0]
    ks = jax.lax.broadcasted_iota(jnp.int32, (K, 1), 0).astype(jnp.float32)
    mv = ks < cnt_ref[...]
    mvb = mv.astype(jnp.bfloat16)
    iv = idx_ref[...].astype(jnp.int16)
    rows = (jax.lax.broadcasted_iota(jnp.int32, (r, 1), 0) + rb * r
            ).astype(jnp.int16)
    zero = jnp.zeros((r, N), jnp.bfloat16)
    acc = zero
    for k in range(K):
        acc = acc + jnp.where(iv[k : k + 1, :] == rows, mvb[k : k + 1, :],
                              zero)
    h_ref[...] = acc
    degd_ref[...] = jnp.sum(acc.astype(jnp.float32), axis=1, keepdims=True)

    @pl.when(rb == 0)
    def _():
        degb = jnp.sum(mv.astype(jnp.float32), axis=0, keepdims=True)
        binv_ref[...] = jnp.where(degb > 0, 1.0 / jnp.maximum(degb, 1e-9), 0.0)


def _build(cnt, idx):
    R = 512
    grid = (N // R,)
    return pl.pallas_call(
        _build_body,
        grid=grid,
        in_specs=[
            pl.BlockSpec((1, N), lambda i: (0, 0)),
            pl.BlockSpec((K, N), lambda i: (0, 0)),
        ],
        out_specs=[
            pl.BlockSpec((R, N), lambda i: (i, 0)),
            pl.BlockSpec((R, 1), lambda i: (i, 0)),
            pl.BlockSpec((1, N), lambda i: (0, 0)),
        ],
        out_shape=[
            jax.ShapeDtypeStruct((N, N), jnp.bfloat16),
            jax.ShapeDtypeStruct((N, 1), jnp.float32),
            jax.ShapeDtypeStruct((1, N), jnp.float32),
        ],
        compiler_params=pltpu.CompilerParams(
            dimension_semantics=("arbitrary",)
        ),
    )(cnt, idx)


def _agg_t_body(h_ref, x_ref, out_ref):
    kb = pl.program_id(1)
    prod = jax.lax.dot_general(
        h_ref[...].astype(jnp.float32), x_ref[...], (((0,), (0,)), ((), ())),
        preferred_element_type=jnp.float32,
    )

    @pl.when(kb == 0)
    def _():
        out_ref[...] = prod

    @pl.when(kb != 0)
    def _():
        out_ref[...] += prod


def _agg_t(H, x):
    F = x.shape[1]
    J = 1024
    R = 1024
    grid = (N // J, N // R)
    return pl.pallas_call(
        _agg_t_body,
        grid=grid,
        in_specs=[
            pl.BlockSpec((R, J), lambda j, k: (k, j)),
            pl.BlockSpec((R, F), lambda j, k: (k, 0)),
        ],
        out_specs=pl.BlockSpec((J, F), lambda j, k: (j, 0)),
        out_shape=jax.ShapeDtypeStruct((N, F), jnp.float32),
        compiler_params=pltpu.CompilerParams(
            dimension_semantics=("parallel", "arbitrary")
        ),
    )(H, x)


def _scatter_body(h_ref, he_ref, binv_ref, degd_ref, bias_ref, out_ref, *,
                  nk, final_tanh):
    kb = pl.program_id(1)
    hb = h_ref[...].astype(jnp.float32) * binv_ref[...]
    prod = jnp.dot(hb, he_ref[...], preferred_element_type=jnp.float32)

    @pl.when(kb == 0)
    def _():
        out_ref[...] = prod

    @pl.when(kb != 0)
    def _():
        out_ref[...] += prod

    @pl.when(kb == nk - 1)
    def _():
        dv = degd_ref[...]
        dinv = jnp.where(dv > 0, 1.0 / jnp.maximum(dv, 1e-9), 0.0)
        r = out_ref[...] * dinv + bias_ref[...]
        out_ref[...] = jnp.tanh(r) if final_tanh else r


def _scatter(H, he, binv, degd, bias, final_tanh):
    F = he.shape[1]
    R = 1024
    J = 1024
    nk = N // J
    grid = (N // R, nk)
    return pl.pallas_call(
        functools.partial(_scatter_body, nk=nk, final_tanh=final_tanh),
        grid=grid,
        in_specs=[
            pl.BlockSpec((R, J), lambda i, k: (i, k)),
            pl.BlockSpec((J, F), lambda i, k: (k, 0)),
            pl.BlockSpec((1, J), lambda i, k: (0, k)),
            pl.BlockSpec((R, 1), lambda i, k: (i, 0)),
            pl.BlockSpec((1, F), lambda i, k: (0, 0)),
        ],
        out_specs=pl.BlockSpec((R, F), lambda i, k: (i, 0)),
        out_shape=jax.ShapeDtypeStruct((N, F), jnp.float32),
        compiler_params=pltpu.CompilerParams(
            dimension_semantics=("parallel", "arbitrary")
        ),
    )(H, he, binv, degd, bias)


def _mlp_body(z_ref, w1_ref, b1_ref, w2_ref, out_ref):
    mid = jnp.dot(z_ref[...], w1_ref[...], preferred_element_type=jnp.float32)
    mid = jnp.maximum(mid + b1_ref[...], 0.0)
    out_ref[...] = jnp.dot(mid, w2_ref[...], preferred_element_type=jnp.float32)


def _mlp(z, W1, b1, W2):
    IN_F, HID = W1.shape
    CODE = W2.shape[1]
    R = 512
    grid = (N // R,)
    return pl.pallas_call(
        _mlp_body,
        grid=grid,
        in_specs=[
            pl.BlockSpec((R, IN_F), lambda i: (i, 0)),
            pl.BlockSpec((IN_F, HID), lambda i: (0, 0)),
            pl.BlockSpec((1, HID), lambda i: (0, 0)),
            pl.BlockSpec((HID, CODE), lambda i: (0, 0)),
        ],
        out_specs=pl.BlockSpec((R, CODE), lambda i: (i, 0)),
        out_shape=jax.ShapeDtypeStruct((N, CODE), jnp.float32),
        compiler_params=pltpu.CompilerParams(
            dimension_semantics=("arbitrary",)
        ),
    )(z, W1, b1, W2)


def _prep_body(cnt_ref, idx_ref, safe_ref, binv_ref):
    ks = jax.lax.broadcasted_iota(jnp.int32, (K, 1), 0).astype(jnp.float32)
    mv = ks < cnt_ref[...]
    safe = jnp.where(mv, idx_ref[...], N)
    safe_ref[...] = jnp.transpose(safe, (1, 0))
    degb = jnp.sum(mv.astype(jnp.float32), axis=0, keepdims=True)
    binv_ref[...] = jnp.where(degb > 0, 1.0 / jnp.maximum(degb, 1e-9), 0.0)


def _prep(cnt, idx):
    C = 512
    grid = (N // C,)
    return pl.pallas_call(
        _prep_body,
        grid=grid,
        in_specs=[
            pl.BlockSpec((1, C), lambda j: (0, j)),
            pl.BlockSpec((K, C), lambda j: (0, j)),
        ],
        out_specs=[
            pl.BlockSpec((C, K), lambda j: (j, 0)),
            pl.BlockSpec((1, C), lambda j: (0, j)),
        ],
        out_shape=[
            jax.ShapeDtypeStruct((N, K), jnp.int32),
            jax.ShapeDtypeStruct((1, N), jnp.float32),
        ],
        compiler_params=pltpu.CompilerParams(
            dimension_semantics=("arbitrary",)
        ),
    )(cnt, idx)


_NW = 32
_JPW = N // _NW


_JPG = 8
_NG = _JPW // _JPG


def _sc_conv(t_pad, safe_flat, safe_grp, binv, zeros, F):
    mesh = plsc.VectorSubcoreMesh(core_axis_name="c", subcore_axis_name="s")

    @functools.partial(
        pl.kernel,
        out_type=[
            jax.ShapeDtypeStruct((N, F), jnp.float32),
            jax.ShapeDtypeStruct((N, F), jnp.float32),
        ],
        mesh=mesh,
        scratch_types=[
            pltpu.VMEM((_JPW * K,), jnp.int32),
            pltpu.VMEM((_NG, _JPG * K), jnp.int32),
            pltpu.VMEM((1, _JPW + 16), jnp.float32),
            pltpu.VMEM((_JPG * K, F), jnp.float32),
            pltpu.VMEM((_JPG * K, F), jnp.float32),
            pltpu.VMEM((N // 16, F), jnp.float32),
            pltpu.VMEM_SHARED((N + 1, F), jnp.float32),
        ],
    )
    def body(t_hbm, safe1_hbm, safe2_hbm, binv_hbm, zeros_hbm, out_a, out_b,
             safe1_v, safe2_v, binv_v, rows_v, sbuf_v, stage_v, acc_sh):
        cid = lax.axis_index("c")
        sid = lax.axis_index("s")
        wid = cid * 16 + sid
        j0 = wid * _JPW

        @pl.when(sid == 0)
        def _():
            pltpu.sync_copy(zeros_hbm, acc_sh)

        pltpu.sync_copy(safe1_hbm.at[pl.ds(j0 * K, _JPW * K)], safe1_v)
        pltpu.sync_copy(safe2_hbm.at[pl.ds(wid * _NG, _NG)], safe2_v)
        pltpu.sync_copy(binv_hbm.at[:, pl.ds(j0, _JPW)],
                        binv_v.at[:, pl.ds(0, _JPW)])
        plsc.subcore_barrier()

        def step(g, carry):
            pltpu.sync_copy(t_hbm.at[safe1_v.at[pl.ds(g * (_JPG * K), _JPG * K)]],
                            rows_v)
            for q in range(_JPG):
                bv = binv_v[0, pl.ds(g * _JPG + q, 16)][0]
                for c in range(F // 16):
                    sl = pl.ds(c * 16, 16)
                    terms = [rows_v[q * K + kk, sl] for kk in range(K)]
                    while len(terms) > 1:
                        terms = [terms[i] + terms[i + 1]
                                 for i in range(0, len(terms), 2)]
                    acc = terms[0] * bv
                    for kk in range(K):
                        sbuf_v[q * K + kk, sl] = acc
            pltpu.sync_copy(sbuf_v, acc_sh.at[safe2_v.at[g]], add=True)
            return carry

        lax.fori_loop(0, _NG, step, 0)
        plsc.subcore_barrier()
        r0 = sid * (N // 16)
        pltpu.sync_copy(acc_sh.at[pl.ds(r0, N // 16)], stage_v)

        @pl.when(cid == 0)
        def _():
            pltpu.sync_copy(stage_v, out_a.at[pl.ds(r0, N // 16)])

        @pl.when(cid == 1)
        def _():
            pltpu.sync_copy(stage_v, out_b.at[pl.ds(r0, N // 16)])

    return body(t_pad, safe_flat, safe_grp, binv, zeros)


def _finish_body(a_ref, b_ref, degd_ref, bias_ref, out_ref):
    F = out_ref.shape[1]
    dv = degd_ref[...]
    dinv = jnp.where(dv > 0, 1.0 / jnp.maximum(dv, 1e-9), 0.0)
    s = a_ref[:, :F] + b_ref[:, :F]
    out_ref[...] = jnp.tanh(s * dinv + bias_ref[...])


def _finish(za, zb, degd, bias):
    Fp = za.shape[1]
    F = bias.shape[1]
    R = 1024
    grid = (N // R,)
    return pl.pallas_call(
        _finish_body,
        grid=grid,
        in_specs=[
            pl.BlockSpec((R, Fp), lambda i: (i, 0)),
            pl.BlockSpec((R, Fp), lambda i: (i, 0)),
            pl.BlockSpec((R, 1), lambda i: (i, 0)),
            pl.BlockSpec((1, F), lambda i: (0, 0)),
        ],
        out_specs=pl.BlockSpec((R, F), lambda i: (i, 0)),
        out_shape=jax.ShapeDtypeStruct((N, F), jnp.float32),
        compiler_params=pltpu.CompilerParams(
            dimension_semantics=("arbitrary",)
        ),
    )(za, zb, degd, bias)


def kernel(x, S, W1, b1, W2, b2):
    idx, cnt = _topk(S)
    H, degd, binv = _build(cnt, idx)
    zero_b = jnp.zeros((1, x.shape[1]), jnp.float32)
    he = _agg_t(H, x)
    z = _scatter(H, he, binv, degd, zero_b, final_tanh=False)
    t = _mlp(z, W1, b1.reshape(1, -1), W2)
    t_pad = jnp.pad(t, ((0, 1), (0, 128 - t.shape[1])))
    zeros = jnp.zeros((N + 1, 128), jnp.float32)
    safeT, binv_p = _prep(cnt, idx)
    safe_flat = safeT.reshape(-1)
    safe_grp = safeT.reshape(N * K // (_JPG * K), _JPG * K)
    z2a, z2b = _sc_conv(t_pad, safe_flat, safe_grp, binv_p, zeros, 128)
    code = _finish(z2a, z2b, degd, b2.reshape(1, -1))
    return code
